# baseline (device time: 16284 ns/iter reference)
import os

import jax
import jax.numpy as jnp
from jax import lax
from jax.experimental import pallas as pl
from jax.experimental.pallas import tpu as pltpu

N_DEV = 16
N_PLANE = 4
_VARIANT = os.environ.get("KVARIANT", "full")


def kernel(x, w_mat):
    m_dim, blk = x.shape
    k_dim = w_mat.shape[0]
    n_dim = w_mat.shape[1]

    sync = {
        "full": "fullbar", "fullbar": "fullbar", "vplane": "plane",
        "vpeer": "peer", "ringbar": "ring", "nobar": "self",
        "nocompute": "fullbar", "nocomm": "fullbar", "barrier": "fullbar",
        "selfbar": "self", "empty": None,
        "vfar": "fullbar", "vfinal": "fullbar", "vout": "fullbar",
        "vrecv": "fullbar",
    }[_VARIANT]
    do_comm = _VARIANT in (
        "full", "fullbar", "vplane", "vpeer", "nobar", "nocompute", "vfar",
        "vfinal", "vout", "vrecv",
    )
    do_compute = _VARIANT in (
        "full", "fullbar", "vplane", "vpeer", "nobar", "nocomm", "vfar",
        "vfinal", "vout", "vrecv",
    )
    if _VARIANT in ("vfar", "vfinal"):
        send_order = sorted(range(1, N_DEV), key=lambda k: -min(k, N_DEV - k))
        recv_order = sorted(range(1, N_DEV), key=lambda k: min(k, N_DEV - k))
    elif _VARIANT == "vrecv":
        send_order = list(range(1, N_DEV))
        recv_order = sorted(range(1, N_DEV), key=lambda k: min(k, N_DEV - k))
    else:
        send_order = list(range(1, N_DEV))
        recv_order = list(range(1, N_DEV))
    hbm_out = _VARIANT in ("vfinal", "vout")

    def body(x_ref, w_ref, out_ref, xblks_ref, acc_ref, send_sems, recv_sems,
             credit_sems, out_copy_sem):
        me = lax.axis_index("i")
        my_plane = me // N_PLANE

        if sync == "fullbar":
            barrier_sem = pltpu.get_barrier_semaphore()
            for k in range(1, N_DEV):
                nbr = lax.rem(me + k, N_DEV)
                pl.semaphore_signal(
                    barrier_sem, inc=1,
                    device_id=(nbr,), device_id_type=pl.DeviceIdType.MESH,
                )
            pl.semaphore_wait(barrier_sem, N_DEV - 1)
        elif sync == "ring":
            barrier_sem = pltpu.get_barrier_semaphore()
            for nbr in (lax.rem(me + 1, N_DEV), lax.rem(me + N_DEV - 1, N_DEV)):
                pl.semaphore_signal(
                    barrier_sem, inc=1,
                    device_id=(nbr,), device_id_type=pl.DeviceIdType.MESH,
                )
            pl.semaphore_wait(barrier_sem, 2)
        elif sync is not None:
            barrier_sem = pltpu.get_barrier_semaphore()
            pl.semaphore_signal(barrier_sem, inc=1)
            pl.semaphore_wait(barrier_sem, 1)

        if sync == "plane":
            pl.semaphore_signal(credit_sems.at[my_plane], inc=1)
            for k in range(1, N_DEV):
                q = lax.rem(me + k, N_DEV)
                pl.semaphore_signal(
                    credit_sems.at[my_plane], inc=1,
                    device_id=(q,), device_id_type=pl.DeviceIdType.MESH,
                )
        elif sync == "peer":
            for k in range(1, N_DEV):
                q = lax.rem(me + k, N_DEV)
                pl.semaphore_signal(
                    credit_sems.at[me], inc=1,
                    device_id=(q,), device_id_type=pl.DeviceIdType.MESH,
                )

        xblks_ref[me] = x_ref[pl.ds(me * blk, blk), :]

        sends = []

        def start_send(dst):
            rdma = pltpu.make_async_remote_copy(
                src_ref=x_ref.at[pl.ds(dst * blk, blk), :],
                dst_ref=xblks_ref.at[me],
                send_sem=send_sems.at[dst],
                recv_sem=recv_sems.at[me],
                device_id=(dst,),
                device_id_type=pl.DeviceIdType.MESH,
            )
            rdma.start()
            sends.append(rdma)

        if do_comm:
            if sync == "plane":
                r_me = lax.rem(me, N_PLANE)
                for delta in range(N_PLANE):
                    p = lax.rem(my_plane + delta, N_PLANE)
                    pl.semaphore_wait(credit_sems.at[p], N_PLANE)
                    for s in range(1 if delta == 0 else 0, N_PLANE):
                        start_send(p * N_PLANE + lax.rem(r_me + s, N_PLANE))

            elif sync == "peer":
                for k in range(1, N_DEV):
                    dst = lax.rem(me + k, N_DEV)
                    pl.semaphore_wait(credit_sems.at[dst], 1)
                    start_send(dst)
            else:
                for k in send_order:
                    start_send(lax.rem(me + k, N_DEV))

            for k in recv_order:
                j = lax.rem(me - k + N_DEV, N_DEV)
                recv = pltpu.make_async_remote_copy(
                    src_ref=x_ref.at[pl.ds(0, blk), :],
                    dst_ref=xblks_ref.at[j],
                    send_sem=send_sems.at[j],
                    recv_sem=recv_sems.at[j],
                    device_id=(j,),
                    device_id_type=pl.DeviceIdType.MESH,
                )
                recv.wait_recv()

        if do_compute:
            xrow = jnp.transpose(xblks_ref[...], (1, 0, 2)).reshape(blk, k_dim)
            acc = jnp.dot(xrow, w_ref[...], preferred_element_type=jnp.float32)

        for rdma in sends:
            rdma.wait_send()

        if do_compute:
            c = 0.7978845608028654
            y = 0.5 * acc * (1.0 + jnp.tanh(c * (acc + 0.044715 * acc * acc * acc)))
        else:
            y = jnp.zeros((blk, n_dim), jnp.float32)
        if hbm_out:
            acc_ref[...] = y
            out_copy = pltpu.make_async_copy(acc_ref, out_ref, out_copy_sem)
            out_copy.start()
            out_copy.wait()
        else:
            out_ref[...] = y

    return pl.pallas_call(
        body,
        out_shape=jax.ShapeDtypeStruct((blk, n_dim), jnp.float32),
        in_specs=[
            pl.BlockSpec(memory_space=pltpu.VMEM),
            pl.BlockSpec(memory_space=pltpu.VMEM),
        ],
        out_specs=pl.BlockSpec(
            memory_space=pl.ANY if hbm_out else pltpu.VMEM
        ),
        scratch_shapes=[
            pltpu.VMEM((N_DEV, blk, blk), jnp.float32),
            pltpu.VMEM((blk, n_dim), jnp.float32),
            pltpu.SemaphoreType.DMA((N_DEV,)),
            pltpu.SemaphoreType.DMA((N_DEV,)),
            pltpu.SemaphoreType.REGULAR((N_DEV,)),
            pltpu.SemaphoreType.DMA,
        ],
        compiler_params=(
            pltpu.CompilerParams()
            if _VARIANT == "empty"
            else pltpu.CompilerParams(collective_id=0)
        ),
    )(x, w_mat)
